# baseline (speedup 1.0000x reference)
import functools

import jax
import jax.numpy as jnp
from jax import lax
from jax.experimental import pallas as pl
from jax.experimental.pallas import tpu as pltpu
from jax.experimental.pallas import tpu_sc as plsc

N = 100000
D = 128
G = 256
NC = 2
NS = 16
NW = NC * NS
CHUNK = 125
CHUNKS_W = 13
ROWS_PER_W = CHUNK * CHUNKS_W
N_SC = NW * ROWS_PER_W
N_TC = N - N_SC
BT = 1000
NBT = N_TC // BT
G_PER_SUB = G // NS

_mesh = plsc.VectorSubcoreMesh(core_axis_name="c", subcore_axis_name="s")


@functools.partial(
    pl.kernel,
    out_type=jax.ShapeDtypeStruct((NC, G, D), jnp.float32),
    mesh=_mesh,
    scratch_types=[
        pltpu.VMEM((CHUNKS_W, CHUNK), jnp.int32),
        pltpu.VMEM((CHUNK, D), jnp.float32),
        pltpu.VMEM((CHUNK, D), jnp.float32),
        pltpu.VMEM((G_PER_SUB, D), jnp.float32),
        pltpu.VMEM_SHARED((G, D), jnp.float32),
        pltpu.SemaphoreType.DMA,
        pltpu.SemaphoreType.DMA,
        pltpu.SemaphoreType.DMA,
        pltpu.SemaphoreType.DMA,
    ],
    compiler_params=pltpu.CompilerParams(use_tc_tiling_on_sc=False),
)
def _segsum_sc(
    feat_hbm, ids_hbm, out_hbm, ids_v, buf0, buf1, zbuf, acc_sh,
    gsem0, gsem1, ssem0, ssem1,
):
    c = lax.axis_index("c")
    s = lax.axis_index("s")
    w = c * NS + s
    base = w * ROWS_PER_W

    def feat_at(j):
        return feat_hbm.at[pl.ds(base + j * CHUNK, CHUNK), :]

    zeros = jnp.zeros((16,), jnp.float32)
    for r in range(G_PER_SUB):
        for d in range(D // 16):
            zbuf[r, pl.ds(d * 16, 16)] = zeros
    pltpu.sync_copy(zbuf, acc_sh.at[pl.ds(s * G_PER_SUB, G_PER_SUB)])

    pltpu.sync_copy(ids_hbm.at[pl.ds(w * CHUNKS_W, CHUNKS_W)], ids_v)
    plsc.subcore_barrier()

    pltpu.async_copy(feat_at(0), buf0, gsem0)
    pltpu.async_copy(feat_at(1), buf1, gsem1)

    npair = CHUNKS_W // 2

    def body(i, carry):
        j0 = 2 * i
        j1 = j0 + 1
        pltpu.make_async_copy(feat_at(j0), buf0, gsem0).wait()
        sc0 = pltpu.async_copy(buf0, acc_sh.at[ids_v.at[j0]], ssem0, add=True)
        pltpu.make_async_copy(feat_at(j1), buf1, gsem1).wait()
        sc1 = pltpu.async_copy(buf1, acc_sh.at[ids_v.at[j1]], ssem1, add=True)
        sc0.wait()

        @pl.when(j0 + 2 < CHUNKS_W)
        def _():
            pltpu.async_copy(feat_at(j0 + 2), buf0, gsem0)

        sc1.wait()

        @pl.when(j1 + 2 < CHUNKS_W)
        def _():
            pltpu.async_copy(feat_at(j1 + 2), buf1, gsem1)

        return carry

    lax.fori_loop(0, npair, body, 0)

    last = CHUNKS_W - 1
    pltpu.make_async_copy(feat_at(last), buf0, gsem0).wait()
    pltpu.sync_copy(buf0, acc_sh.at[ids_v.at[last]], add=True)

    plsc.subcore_barrier()
    pltpu.sync_copy(
        acc_sh.at[pl.ds(s * G_PER_SUB, G_PER_SUB)],
        out_hbm.at[c, pl.ds(s * G_PER_SUB, G_PER_SUB), :],
    )


def _tc_body(ids_ref, feat_ref, o_ref):
    i = pl.program_id(0)
    blk = feat_ref[...]
    idb = ids_ref[0, 0, :]
    onehot = (
        lax.broadcasted_iota(jnp.int32, (BT, G), 1) == idb[:, None]
    ).astype(jnp.bfloat16)
    part = lax.dot_general(
        onehot, blk.astype(jnp.bfloat16), (((0,), (0,)), ((), ())),
        preferred_element_type=jnp.float32,
    )

    @pl.when(i == 0)
    def _():
        o_ref[...] = part

    @pl.when(i > 0)
    def _():
        o_ref[...] += part


_tc_segsum = pl.pallas_call(
    _tc_body,
    grid=(NBT,),
    in_specs=[
        pl.BlockSpec((1, 1, BT), lambda i: (N_SC // BT + i, 0, 0)),
        pl.BlockSpec((BT, D), lambda i: (N_SC // BT + i, 0)),
    ],
    out_specs=pl.BlockSpec((G, D), lambda i: (0, 0)),
    out_shape=jax.ShapeDtypeStruct((G, D), jnp.float32),
)


def _combine_body(p_ref, t_ref, o_ref):
    o_ref[...] = p_ref[0] + p_ref[1] + t_ref[...]


_combine = pl.pallas_call(
    _combine_body,
    out_shape=jax.ShapeDtypeStruct((G, D), jnp.float32),
)


def kernel(feat, segment_ids, num_segments):
    del num_segments
    ids = segment_ids.astype(jnp.int32)
    ids_sc = ids[:N_SC].reshape(NW * CHUNKS_W, CHUNK)
    ids_tc = ids.reshape(N // BT, 1, BT)
    sc_partials = _segsum_sc(feat, ids_sc)
    tc_partial = _tc_segsum(ids_tc, feat)
    return _combine(sc_partials, tc_partial)


# speedup vs baseline: 1.2512x; 1.2512x over previous
import functools

import jax
import jax.numpy as jnp
from jax import lax
from jax.experimental import pallas as pl
from jax.experimental.pallas import tpu as pltpu
from jax.experimental.pallas import tpu_sc as plsc

N = 100000
D = 128
G = 256
NC = 2
NS = 16
NW = NC * NS
CHUNK = 125
CHUNKS_W = 17
ROWS_PER_W = CHUNK * CHUNKS_W
N_SC = NW * ROWS_PER_W
N_TC = N - N_SC
BT = 2000
NBT = N_TC // BT
G_PER_SUB = G // NS

_mesh = plsc.VectorSubcoreMesh(core_axis_name="c", subcore_axis_name="s")


@functools.partial(
    pl.kernel,
    out_type=jax.ShapeDtypeStruct((NC, G, D), jnp.float32),
    mesh=_mesh,
    scratch_types=[
        pltpu.VMEM((CHUNKS_W, CHUNK), jnp.int32),
        pltpu.VMEM((CHUNK, D), jnp.float32),
        pltpu.VMEM((CHUNK, D), jnp.float32),
        pltpu.VMEM((G_PER_SUB, D), jnp.float32),
        pltpu.VMEM_SHARED((G, D), jnp.float32),
        pltpu.SemaphoreType.DMA,
        pltpu.SemaphoreType.DMA,
        pltpu.SemaphoreType.DMA,
        pltpu.SemaphoreType.DMA,
    ],
    compiler_params=pltpu.CompilerParams(use_tc_tiling_on_sc=False),
)
def _segsum_sc(
    feat_hbm, ids_hbm, out_hbm, ids_v, buf0, buf1, zbuf, acc_sh,
    gsem0, gsem1, ssem0, ssem1,
):
    c = lax.axis_index("c")
    s = lax.axis_index("s")
    w = c * NS + s
    base = w * ROWS_PER_W

    def feat_at(j):
        return feat_hbm.at[pl.ds(base + j * CHUNK, CHUNK), :]

    zeros = jnp.zeros((16,), jnp.float32)
    for r in range(G_PER_SUB):
        for d in range(D // 16):
            zbuf[r, pl.ds(d * 16, 16)] = zeros
    pltpu.sync_copy(zbuf, acc_sh.at[pl.ds(s * G_PER_SUB, G_PER_SUB)])

    pltpu.sync_copy(ids_hbm.at[pl.ds(w * CHUNKS_W, CHUNKS_W)], ids_v)
    plsc.subcore_barrier()

    pltpu.async_copy(feat_at(0), buf0, gsem0)
    pltpu.async_copy(feat_at(1), buf1, gsem1)

    npair = CHUNKS_W // 2

    def body(i, carry):
        j0 = 2 * i
        j1 = j0 + 1
        pltpu.make_async_copy(feat_at(j0), buf0, gsem0).wait()
        sc0 = pltpu.async_copy(buf0, acc_sh.at[ids_v.at[j0]], ssem0, add=True)
        pltpu.make_async_copy(feat_at(j1), buf1, gsem1).wait()
        sc1 = pltpu.async_copy(buf1, acc_sh.at[ids_v.at[j1]], ssem1, add=True)
        sc0.wait()

        @pl.when(j0 + 2 < CHUNKS_W)
        def _():
            pltpu.async_copy(feat_at(j0 + 2), buf0, gsem0)

        sc1.wait()

        @pl.when(j1 + 2 < CHUNKS_W)
        def _():
            pltpu.async_copy(feat_at(j1 + 2), buf1, gsem1)

        return carry

    lax.fori_loop(0, npair, body, 0)

    last = CHUNKS_W - 1
    pltpu.make_async_copy(feat_at(last), buf0, gsem0).wait()
    pltpu.sync_copy(buf0, acc_sh.at[ids_v.at[last]], add=True)

    plsc.subcore_barrier()
    pltpu.sync_copy(
        acc_sh.at[pl.ds(s * G_PER_SUB, G_PER_SUB)],
        out_hbm.at[c, pl.ds(s * G_PER_SUB, G_PER_SUB), :],
    )


def _tc_body(ids_ref, feat_ref, o_ref):
    i = pl.program_id(0)
    blk = feat_ref[...]
    idb = ids_ref[0, 0, :]
    onehot = (
        lax.broadcasted_iota(jnp.int32, (BT, G), 1) == idb[:, None]
    ).astype(jnp.float32)
    part = lax.dot_general(
        onehot, blk, (((0,), (0,)), ((), ())),
        preferred_element_type=jnp.float32,
    )

    @pl.when(i == 0)
    def _():
        o_ref[...] = part

    @pl.when(i > 0)
    def _():
        o_ref[...] += part


_tc_segsum = pl.pallas_call(
    _tc_body,
    grid=(NBT,),
    in_specs=[
        pl.BlockSpec((1, 1, BT), lambda i: (N_SC // BT + i, 0, 0)),
        pl.BlockSpec((BT, D), lambda i: (N_SC // BT + i, 0)),
    ],
    out_specs=pl.BlockSpec((G, D), lambda i: (0, 0)),
    out_shape=jax.ShapeDtypeStruct((G, D), jnp.float32),
)


def _combine_body(p_ref, t_ref, o_ref):
    o_ref[...] = p_ref[0] + p_ref[1] + t_ref[...]


_combine = pl.pallas_call(
    _combine_body,
    out_shape=jax.ShapeDtypeStruct((G, D), jnp.float32),
)


def kernel(feat, segment_ids, num_segments):
    del num_segments
    ids = segment_ids.astype(jnp.int32)
    ids_sc = ids[:N_SC].reshape(NW * CHUNKS_W, CHUNK)
    ids_tc = ids.reshape(N // BT, 1, BT)
    sc_partials = _segsum_sc(feat, ids_sc)
    tc_partial = _tc_segsum(ids_tc, feat)
    return _combine(sc_partials, tc_partial)


# speedup vs baseline: 1.2544x; 1.0026x over previous
import functools

import jax
import jax.numpy as jnp
from jax import lax
from jax.experimental import pallas as pl
from jax.experimental.pallas import tpu as pltpu
from jax.experimental.pallas import tpu_sc as plsc

N = 100000
D = 128
G = 256
NC = 2
NS = 16
NW = NC * NS
CHUNK = 125
CHUNKS_W = 17
ROWS_PER_W = CHUNK * CHUNKS_W
N_SC = NW * ROWS_PER_W
N_TC = N - N_SC
BT = 1000
NBT = N_TC // BT
G_PER_SUB = G // NS

_mesh = plsc.VectorSubcoreMesh(core_axis_name="c", subcore_axis_name="s")


@functools.partial(
    pl.kernel,
    out_type=jax.ShapeDtypeStruct((NC, G, D), jnp.float32),
    mesh=_mesh,
    scratch_types=[
        pltpu.VMEM((CHUNKS_W, CHUNK), jnp.int32),
        pltpu.VMEM((CHUNK, D), jnp.float32),
        pltpu.VMEM((CHUNK, D), jnp.float32),
        pltpu.VMEM((G_PER_SUB, D), jnp.float32),
        pltpu.VMEM_SHARED((G, D), jnp.float32),
        pltpu.SemaphoreType.DMA,
        pltpu.SemaphoreType.DMA,
        pltpu.SemaphoreType.DMA,
        pltpu.SemaphoreType.DMA,
    ],
    compiler_params=pltpu.CompilerParams(use_tc_tiling_on_sc=False),
)
def _segsum_sc(
    feat_hbm, ids_hbm, out_hbm, ids_v, buf0, buf1, zbuf, acc_sh,
    gsem0, gsem1, ssem0, ssem1,
):
    c = lax.axis_index("c")
    s = lax.axis_index("s")
    w = c * NS + s
    base = w * ROWS_PER_W

    def feat_at(j):
        return feat_hbm.at[pl.ds(base + j * CHUNK, CHUNK), :]

    zeros = jnp.zeros((16,), jnp.float32)
    for r in range(G_PER_SUB):
        for d in range(D // 16):
            zbuf[r, pl.ds(d * 16, 16)] = zeros
    pltpu.sync_copy(zbuf, acc_sh.at[pl.ds(s * G_PER_SUB, G_PER_SUB)])

    pltpu.sync_copy(ids_hbm.at[pl.ds(w * CHUNKS_W, CHUNKS_W)], ids_v)
    plsc.subcore_barrier()

    pltpu.async_copy(feat_at(0), buf0, gsem0)
    pltpu.async_copy(feat_at(1), buf1, gsem1)

    npair = CHUNKS_W // 2

    def body(i, carry):
        j0 = 2 * i
        j1 = j0 + 1
        pltpu.make_async_copy(feat_at(j0), buf0, gsem0).wait()
        sc0 = pltpu.async_copy(buf0, acc_sh.at[ids_v.at[j0]], ssem0, add=True)
        pltpu.make_async_copy(feat_at(j1), buf1, gsem1).wait()
        sc1 = pltpu.async_copy(buf1, acc_sh.at[ids_v.at[j1]], ssem1, add=True)
        sc0.wait()

        @pl.when(j0 + 2 < CHUNKS_W)
        def _():
            pltpu.async_copy(feat_at(j0 + 2), buf0, gsem0)

        sc1.wait()

        @pl.when(j1 + 2 < CHUNKS_W)
        def _():
            pltpu.async_copy(feat_at(j1 + 2), buf1, gsem1)

        return carry

    lax.fori_loop(0, npair, body, 0)

    last = CHUNKS_W - 1
    pltpu.make_async_copy(feat_at(last), buf0, gsem0).wait()
    pltpu.sync_copy(buf0, acc_sh.at[ids_v.at[last]], add=True)

    plsc.subcore_barrier()
    pltpu.sync_copy(
        acc_sh.at[pl.ds(s * G_PER_SUB, G_PER_SUB)],
        out_hbm.at[c, pl.ds(s * G_PER_SUB, G_PER_SUB), :],
    )


def _tc_body(ids_ref, feat_ref, o_ref):
    i = pl.program_id(0)
    blk = feat_ref[...]
    idb = ids_ref[0, 0, :]
    onehot = (
        lax.broadcasted_iota(jnp.int32, (BT, G), 1) == idb[:, None]
    ).astype(jnp.float32)
    part = lax.dot_general(
        onehot, blk, (((0,), (0,)), ((), ())),
        preferred_element_type=jnp.float32,
    )

    @pl.when(i == 0)
    def _():
        o_ref[...] = part

    @pl.when(i > 0)
    def _():
        o_ref[...] += part


_tc_segsum = pl.pallas_call(
    _tc_body,
    grid=(NBT,),
    in_specs=[
        pl.BlockSpec((1, 1, BT), lambda i: (N_SC // BT + i, 0, 0)),
        pl.BlockSpec((BT, D), lambda i: (N_SC // BT + i, 0)),
    ],
    out_specs=pl.BlockSpec((G, D), lambda i: (0, 0)),
    out_shape=jax.ShapeDtypeStruct((G, D), jnp.float32),
)


def _combine_body(p_ref, t_ref, o_ref):
    o_ref[...] = p_ref[0] + p_ref[1] + t_ref[...]


_combine = pl.pallas_call(
    _combine_body,
    out_shape=jax.ShapeDtypeStruct((G, D), jnp.float32),
)


def kernel(feat, segment_ids, num_segments):
    del num_segments
    ids = segment_ids.astype(jnp.int32)
    ids_sc = ids[:N_SC].reshape(NW * CHUNKS_W, CHUNK)
    ids_tc = ids.reshape(N // BT, 1, BT)
    sc_partials = _segsum_sc(feat, ids_sc)
    tc_partial = _tc_segsum(ids_tc, feat)
    return _combine(sc_partials, tc_partial)


# speedup vs baseline: 1.2734x; 1.0151x over previous
import functools

import jax
import jax.numpy as jnp
from jax import lax
from jax.experimental import pallas as pl
from jax.experimental.pallas import tpu as pltpu
from jax.experimental.pallas import tpu_sc as plsc

N = 100000
D = 128
G = 256
NC = 2
NS = 16
NW = NC * NS
CHUNK = 125
CHUNKS_W = 17
ROWS_PER_W = CHUNK * CHUNKS_W
N_SC = NW * ROWS_PER_W
N_TC = N - N_SC
BT = 1000
NBT = N_TC // BT
G_PER_SUB = G // NS

_mesh = plsc.VectorSubcoreMesh(core_axis_name="c", subcore_axis_name="s")


@functools.partial(
    pl.kernel,
    out_type=jax.ShapeDtypeStruct((NC, G, D), jnp.float32),
    mesh=_mesh,
    scratch_types=[
        pltpu.VMEM((CHUNKS_W, CHUNK), jnp.int32),
        pltpu.VMEM((CHUNK, D), jnp.float32),
        pltpu.VMEM((CHUNK, D), jnp.float32),
        pltpu.VMEM((G_PER_SUB, D), jnp.float32),
        pltpu.VMEM_SHARED((G, D), jnp.float32),
        pltpu.SemaphoreType.DMA,
        pltpu.SemaphoreType.DMA,
        pltpu.SemaphoreType.DMA,
        pltpu.SemaphoreType.DMA,
    ],
    compiler_params=pltpu.CompilerParams(use_tc_tiling_on_sc=False),
)
def _segsum_sc(
    feat_hbm, ids_hbm, out_hbm, ids_v, buf0, buf1, zbuf, acc_sh,
    gsem0, gsem1, ssem0, ssem1,
):
    c = lax.axis_index("c")
    s = lax.axis_index("s")
    w = c * NS + s
    base = w * ROWS_PER_W

    def feat_at(j):
        return feat_hbm.at[pl.ds(base + j * CHUNK, CHUNK), :]

    pltpu.async_copy(feat_at(0), buf0, gsem0)
    pltpu.async_copy(feat_at(1), buf1, gsem1)

    zeros = jnp.zeros((16,), jnp.float32)
    for r in range(G_PER_SUB):
        for d in range(D // 16):
            zbuf[r, pl.ds(d * 16, 16)] = zeros
    pltpu.sync_copy(zbuf, acc_sh.at[pl.ds(s * G_PER_SUB, G_PER_SUB)])

    plsc.subcore_barrier()
    pltpu.sync_copy(ids_hbm.at[pl.ds(w * CHUNKS_W, CHUNKS_W)], ids_v)

    npair = CHUNKS_W // 2

    def body(i, carry):
        j0 = 2 * i
        j1 = j0 + 1
        pltpu.make_async_copy(feat_at(j0), buf0, gsem0).wait()
        sc0 = pltpu.async_copy(buf0, acc_sh.at[ids_v.at[j0]], ssem0, add=True)
        pltpu.make_async_copy(feat_at(j1), buf1, gsem1).wait()
        sc1 = pltpu.async_copy(buf1, acc_sh.at[ids_v.at[j1]], ssem1, add=True)
        sc0.wait()

        @pl.when(j0 + 2 < CHUNKS_W)
        def _():
            pltpu.async_copy(feat_at(j0 + 2), buf0, gsem0)

        sc1.wait()

        @pl.when(j1 + 2 < CHUNKS_W)
        def _():
            pltpu.async_copy(feat_at(j1 + 2), buf1, gsem1)

        return carry

    lax.fori_loop(0, npair, body, 0)

    last = CHUNKS_W - 1
    pltpu.make_async_copy(feat_at(last), buf0, gsem0).wait()
    pltpu.sync_copy(buf0, acc_sh.at[ids_v.at[last]], add=True)

    plsc.subcore_barrier()
    pltpu.sync_copy(
        acc_sh.at[pl.ds(s * G_PER_SUB, G_PER_SUB)],
        out_hbm.at[c, pl.ds(s * G_PER_SUB, G_PER_SUB), :],
    )


def _tc_body(ids_ref, feat_ref, o_ref):
    i = pl.program_id(0)
    blk = feat_ref[...]
    idb = ids_ref[0, 0, :]
    onehot = (
        lax.broadcasted_iota(jnp.int32, (BT, G), 1) == idb[:, None]
    ).astype(jnp.float32)
    part = lax.dot_general(
        onehot, blk, (((0,), (0,)), ((), ())),
        preferred_element_type=jnp.float32,
    )

    @pl.when(i == 0)
    def _():
        o_ref[...] = part

    @pl.when(i > 0)
    def _():
        o_ref[...] += part


_tc_segsum = pl.pallas_call(
    _tc_body,
    grid=(NBT,),
    in_specs=[
        pl.BlockSpec((1, 1, BT), lambda i: (N_SC // BT + i, 0, 0)),
        pl.BlockSpec((BT, D), lambda i: (N_SC // BT + i, 0)),
    ],
    out_specs=pl.BlockSpec((G, D), lambda i: (0, 0)),
    out_shape=jax.ShapeDtypeStruct((G, D), jnp.float32),
)


def _combine_body(p_ref, t_ref, o_ref):
    o_ref[...] = p_ref[0] + p_ref[1] + t_ref[...]


_combine = pl.pallas_call(
    _combine_body,
    out_shape=jax.ShapeDtypeStruct((G, D), jnp.float32),
)


def kernel(feat, segment_ids, num_segments):
    del num_segments
    ids = segment_ids.astype(jnp.int32)
    ids_sc = ids[:N_SC].reshape(NW * CHUNKS_W, CHUNK)
    ids_tc = ids.reshape(N // BT, 1, BT)
    sc_partials = _segsum_sc(feat, ids_sc)
    tc_partial = _tc_segsum(ids_tc, feat)
    return _combine(sc_partials, tc_partial)
